# baseline (speedup 1.0000x reference)
import functools

import jax
import jax.numpy as jnp
from jax import lax
from jax.experimental import pallas as pl
from jax.experimental.pallas import tpu as pltpu
from jax.experimental.pallas import tpu_sc as plsc

L = 50
LP = 56
D = 300
DP = 384
H = 4
G = 32
NG = 512

_NC = 2
_NS = 16
_NW = _NC * _NS
_NTOK = NG * LP
_BPW = _NTOK // _NW
_CH = 224
_NCHUNK = _BPW // _CH


@functools.lru_cache(maxsize=None)
def _make_gather_sc():
    @functools.partial(
        pl.kernel,
        mesh=plsc.VectorSubcoreMesh(core_axis_name="c", subcore_axis_name="s"),
        out_type=jax.ShapeDtypeStruct((_NTOK, DP), jnp.float32),
        scratch_types=[
            pltpu.VMEM((_CH,), jnp.int32),
            pltpu.VMEM((_CH, DP), jnp.float32),
            pltpu.SemaphoreType.DMA,
        ],
    )
    def _gather_sc(idx_hbm, table_hbm, out_hbm, idx_v, rows_v, sem):
        wid = lax.axis_index("s") * _NC + lax.axis_index("c")
        base = wid * _BPW
        for c in range(_NCHUNK):
            off = base + c * _CH
            pltpu.sync_copy(idx_hbm.at[pl.ds(off, _CH)], idx_v)
            pltpu.async_copy(table_hbm.at[idx_v], rows_v, sem).wait()
            pltpu.sync_copy(rows_v, out_hbm.at[pl.ds(off, _CH)])

    return _gather_sc


def _dot(a, b, dims):
    return jax.lax.dot_general(a, b, dimension_numbers=(dims, ((), ())),
                               preferred_element_type=jnp.float32)


def _round(x2d, eye, rowmask, wsim_gat, wsim_sage,
           Wg, al, ar, bg, Ws, bs, whw, bhw):
    zs = [_dot(x2d, Wg[h], ((1,), (0,))) for h in range(H)]
    e1 = jnp.maximum(x2d * wsim_gat, 0.0)
    e2 = jnp.maximum(x2d * wsim_sage, 0.0)
    ones_col = jnp.ones((LP, 1), jnp.float32)

    outs = []
    for g in range(G):
        s = slice(g * LP, (g + 1) * LP)
        xg = x2d[s]
        e1g = e1[s]
        mgat = (_dot(e1g, e1g, ((1,), (1,))) >= 0.1) | eye
        e2g = e2[s]
        msage = (_dot(e2g, e2g, ((1,), (1,))) >= 0.1) | eye

        chans = []
        for h in range(H):
            z = zs[h][s]
            el = _dot(z, al[h], ((1,), (1,)))
            er = _dot(ar[h], z, ((1,), (1,)))
            e = el + er
            e = jnp.where(e >= 0, e, 0.2 * e)
            e = jnp.where(mgat, e, -1e9)
            emax = jnp.max(e, axis=0, keepdims=True)
            ex = jnp.exp(e - emax)
            alpha = ex / jnp.sum(ex, axis=0, keepdims=True)
            att = _dot(alpha, z, ((0,), (0,)))
            chans.append(att + bg[h])

        m = msage.astype(jnp.float32)
        deg = _dot(m, ones_col, ((0,), (0,)))
        agg = _dot(m, xg, ((0,), (0,)))
        hs = (agg + xg) / (deg + 1.0)
        chans.append(_dot(hs, Ws, ((1,), (0,))) + bs)

        out = xg
        for ch in chans:
            gate = jax.nn.sigmoid(jnp.sum(ch * whw, axis=1, keepdims=True) + bhw)
            out = out + ch * gate
        outs.append(out)

    return jnp.concatenate(outs, axis=0) * rowmask


def _main_kernel(x_ref, ws1_ref, ws2_ref, wws_ref, bws_ref,
                 Wg1, al1, ar1, bg1, Ws1, bs1, whw1, bhw1,
                 Wg2, al2, ar2, bg2, Ws2, bs2, whw2, bhw2,
                 out_ref):
    x2d = x_ref[...].reshape(G * LP, DP)[:, 0:D]
    rowmask = (lax.broadcasted_iota(jnp.int32, (G * LP, 1), 0) % LP
               < L).astype(jnp.float32)
    x2d = x2d * rowmask
    iu = lax.broadcasted_iota(jnp.int32, (LP, LP), 0)
    iv = lax.broadcasted_iota(jnp.int32, (LP, LP), 1)
    eye = iu == iv
    ws1 = ws1_ref[...]
    ws2 = ws2_ref[...]
    x2d = _round(x2d, eye, rowmask, ws2, ws1, Wg1[...], al1[...], ar1[...],
                 bg1[...], Ws1[...], bs1[...], whw1[...], bhw1[...])
    x2d = _round(x2d, eye, rowmask, ws2, ws1, Wg2[...], al2[...], ar2[...],
                 bg2[...], Ws2[...], bs2[...], whw2[...], bhw2[...])
    gate = jax.nn.sigmoid(jnp.sum(x2d * wws_ref[...], axis=1, keepdims=True)
                          + bws_ref[...])
    xw = x2d * gate
    for g in range(G):
        out_ref[g] = jnp.sum(xw[g * LP:(g + 1) * LP], axis=0, keepdims=True)


def _pearson_kernel(p_ref, out_ref):
    p = p_ref[...].reshape(NG, D)
    g1 = p[0:NG // 2, :]
    g2 = p[NG // 2:NG, :]
    g1 = g1 - jnp.mean(g1, axis=1, keepdims=True)
    g2 = g2 - jnp.mean(g2, axis=1, keepdims=True)
    num = jnp.sum(g1 * g2, axis=1)
    den = jnp.sqrt(jnp.sum(g1 * g1, axis=1)) * jnp.sqrt(jnp.sum(g2 * g2, axis=1))
    out_ref[...] = num / den * 5.0


def _mg_args(p):
    Wg = p["W_gat"].reshape(D, H, D).transpose(1, 0, 2)
    al = p["attn_l"][:, None, :]
    ar = p["attn_r"][:, None, :]
    bg = p["b_gat"][:, None, :]
    whw = p["W_hw"].reshape(1, D)
    bhw = p["b_hw"].reshape(1, 1)
    return [Wg, al, ar, bg, p["W_sage"], p["b_sage"].reshape(1, D), whw, bhw]


def kernel(sentence_1, sentence_2, emb_table, w_sim1, w_sim2, params):
    idx2d = jnp.concatenate([sentence_1.T, sentence_2.T], axis=0)
    idx = jnp.pad(idx2d.astype(jnp.int32), ((0, 0), (0, LP - L))).reshape(-1)
    table_p = jnp.pad(emb_table, ((0, 0), (0, DP - D)))
    x = _make_gather_sc()(idx, table_p).reshape(NG // G, G * LP, DP)

    full = lambda shape: pl.BlockSpec(shape, lambda i: (0,) * len(shape))
    w_specs = [full((H, D, D)), full((H, 1, D)), full((H, 1, D)), full((H, 1, D)),
               full((D, D)), full((1, D)), full((1, D)), full((1, 1))]
    grid_spec = pl.GridSpec(
        grid=(NG // G,),
        in_specs=[pl.BlockSpec((1, G * LP, DP), lambda i: (i, 0, 0)),
                  full((1, D)), full((1, D)), full((1, D)), full((1, 1))]
                 + w_specs + w_specs,
        out_specs=pl.BlockSpec((G, 1, D), lambda i: (i, 0, 0)),
    )
    pooled = pl.pallas_call(
        _main_kernel,
        grid_spec=grid_spec,
        out_shape=jax.ShapeDtypeStruct((NG, 1, D), jnp.float32),
    )(x, w_sim1.reshape(1, D), w_sim2.reshape(1, D),
      params["W_ws"].reshape(1, D), params["b_ws"].reshape(1, 1),
      *_mg_args(params["mg1"]), *_mg_args(params["mg2"]))

    return pl.pallas_call(
        _pearson_kernel,
        out_shape=jax.ShapeDtypeStruct((NG // 2,), jnp.float32),
    )(pooled)


# speedup vs baseline: 1.0137x; 1.0137x over previous
import functools

import jax
import jax.numpy as jnp
from jax import lax
from jax.experimental import pallas as pl
from jax.experimental.pallas import tpu as pltpu
from jax.experimental.pallas import tpu_sc as plsc

L = 50
LP = 56
D = 300
DP = 384
H = 4
G = 16
NG = 512

_NC = 2
_NS = 16
_NW = _NC * _NS
_NTOK = NG * LP
_BPW = _NTOK // _NW
_CH = 112
_NCHUNK = _BPW // _CH
_TROWS = 100000 * D // 128


@functools.lru_cache(maxsize=None)
def _make_gather_sc():
    @functools.partial(
        pl.kernel,
        mesh=plsc.VectorSubcoreMesh(core_axis_name="c", subcore_axis_name="s"),
        out_type=jax.ShapeDtypeStruct((3, _NTOK, 128), jnp.float32),
        scratch_types=[
            pltpu.VMEM((_CH,), jnp.int32),
            pltpu.VMEM((_CH,), jnp.int32),
            pltpu.VMEM((4, _CH), jnp.int32),
            pltpu.VMEM((4, _CH, 128), jnp.float32),
            pltpu.VMEM((3, _CH, 128), jnp.float32),
            pltpu.SemaphoreType.DMA,
        ],
        compiler_params=pltpu.CompilerParams(use_tc_tiling_on_sc=False,
                                             needs_layout_passes=False),
    )
    def _gather_sc(idx_hbm, table_hbm, out_hbm, idx_v, o_v, idx4_v, rows_v,
                   outp_v, sem):
        wid = lax.axis_index("s") * _NC + lax.axis_index("c")
        base = wid * _BPW
        lane = lax.iota(jnp.int32, 16)
        for c in range(_NCHUNK):
            off = base + c * _CH
            pltpu.sync_copy(idx_hbm.at[pl.ds(off, _CH)], idx_v)
            for j in range(_CH // 16):
                r = idx_v[pl.ds(j * 16, 16)]
                w0 = r * 300
                s0 = w0 >> 7
                o_v[pl.ds(j * 16, 16)] = w0 & 127
                for k in range(4):
                    idx4_v[k, pl.ds(j * 16, 16)] = s0 + k
            copies = [pltpu.async_copy(table_hbm.at[idx4_v.at[k]],
                                       rows_v.at[k], sem) for k in range(4)]
            for cp in copies:
                cp.wait()

            def body(t, _):
                tv = jnp.full((16,), t, jnp.int32)
                o = plsc.load_gather(o_v, [tv])
                for j in range(19):
                    w = (o + 16 * j) + lane
                    kk = w >> 7
                    cc = w & 127
                    vals = plsc.load_gather(rows_v, [kk, tv, cc])
                    p, c0 = (16 * j) // 128, (16 * j) % 128
                    outp_v[p, t, pl.ds(c0, 16)] = vals
                return 0

            lax.fori_loop(0, _CH, body, 0)
            for p in range(3):
                pltpu.sync_copy(outp_v.at[p], out_hbm.at[p, pl.ds(off, _CH)])

    return _gather_sc


def _dot(a, b, dims):
    return jax.lax.dot_general(a, b, dimension_numbers=(dims, ((), ())),
                               preferred_element_type=jnp.float32)


def _round(x2d, eye, rowmask, wsim_gat, wsim_sage,
           Wg, al, ar, bg, Ws, bs, whw, bhw):
    zs = [_dot(x2d, Wg[h], ((1,), (0,))) for h in range(H)]
    e1 = jnp.maximum(x2d * wsim_gat, 0.0)
    e2 = jnp.maximum(x2d * wsim_sage, 0.0)
    ones_col = jnp.ones((LP, 1), jnp.float32)

    outs = []
    for g in range(G):
        s = slice(g * LP, (g + 1) * LP)
        xg = x2d[s]
        e1g = e1[s]
        mgat = (_dot(e1g, e1g, ((1,), (1,))) >= 0.1) | eye
        e2g = e2[s]
        msage = (_dot(e2g, e2g, ((1,), (1,))) >= 0.1) | eye

        chans = []
        for h in range(H):
            z = zs[h][s]
            el = _dot(z, al[h], ((1,), (1,)))
            er = _dot(ar[h], z, ((1,), (1,)))
            e = el + er
            e = jnp.where(e >= 0, e, 0.2 * e)
            e = jnp.where(mgat, e, -1e9)
            emax = jnp.max(e, axis=0, keepdims=True)
            ex = jnp.exp(e - emax)
            alpha = ex / jnp.sum(ex, axis=0, keepdims=True)
            att = _dot(alpha, z, ((0,), (0,)))
            chans.append(att + bg[h])

        m = msage.astype(jnp.float32)
        deg = _dot(m, ones_col, ((0,), (0,)))
        agg = _dot(m, xg, ((0,), (0,)))
        hs = (agg + xg) / (deg + 1.0)
        chans.append(_dot(hs, Ws, ((1,), (0,))) + bs)

        out = xg
        for ch in chans:
            gate = jax.nn.sigmoid(jnp.sum(ch * whw, axis=1, keepdims=True) + bhw)
            out = out + ch * gate
        outs.append(out)

    return jnp.concatenate(outs, axis=0) * rowmask


def _main_kernel(x_ref, ws1_ref, ws2_ref, wws_ref, bws_ref,
                 Wg1, al1, ar1, bg1, Ws1, bs1, whw1, bhw1,
                 Wg2, al2, ar2, bg2, Ws2, bs2, whw2, bhw2,
                 out_ref):
    x2d = jnp.concatenate([x_ref[0, 0], x_ref[1, 0], x_ref[2, 0][:, 0:44]],
                          axis=1)
    rowmask = (lax.broadcasted_iota(jnp.int32, (G * LP, 1), 0) % LP
               < L).astype(jnp.float32)
    x2d = x2d * rowmask
    iu = lax.broadcasted_iota(jnp.int32, (LP, LP), 0)
    iv = lax.broadcasted_iota(jnp.int32, (LP, LP), 1)
    eye = iu == iv
    ws1 = ws1_ref[...]
    ws2 = ws2_ref[...]
    x2d = _round(x2d, eye, rowmask, ws2, ws1, Wg1[...], al1[...], ar1[...],
                 bg1[...], Ws1[...], bs1[...], whw1[...], bhw1[...])
    x2d = _round(x2d, eye, rowmask, ws2, ws1, Wg2[...], al2[...], ar2[...],
                 bg2[...], Ws2[...], bs2[...], whw2[...], bhw2[...])
    gate = jax.nn.sigmoid(jnp.sum(x2d * wws_ref[...], axis=1, keepdims=True)
                          + bws_ref[...])
    xw = x2d * gate
    for g in range(G):
        out_ref[g] = jnp.sum(xw[g * LP:(g + 1) * LP], axis=0, keepdims=True)


def _pearson_kernel(p_ref, out_ref):
    p = p_ref[...].reshape(NG, D)
    g1 = p[0:NG // 2, :]
    g2 = p[NG // 2:NG, :]
    g1 = g1 - jnp.mean(g1, axis=1, keepdims=True)
    g2 = g2 - jnp.mean(g2, axis=1, keepdims=True)
    num = jnp.sum(g1 * g2, axis=1)
    den = jnp.sqrt(jnp.sum(g1 * g1, axis=1)) * jnp.sqrt(jnp.sum(g2 * g2, axis=1))
    out_ref[...] = num / den * 5.0


def _mg_args(p):
    Wg = p["W_gat"].reshape(D, H, D).transpose(1, 0, 2)
    al = p["attn_l"][:, None, :]
    ar = p["attn_r"][:, None, :]
    bg = p["b_gat"][:, None, :]
    whw = p["W_hw"].reshape(1, D)
    bhw = p["b_hw"].reshape(1, 1)
    return [Wg, al, ar, bg, p["W_sage"], p["b_sage"].reshape(1, D), whw, bhw]


def kernel(sentence_1, sentence_2, emb_table, w_sim1, w_sim2, params):
    idx2d = jnp.concatenate([sentence_1.T, sentence_2.T], axis=0)
    idx = jnp.pad(idx2d.astype(jnp.int32), ((0, 0), (0, LP - L))).reshape(-1)
    table_v = emb_table.reshape(_TROWS, 128)
    x = _make_gather_sc()(idx, table_v).reshape(3, NG // G, G * LP, 128)

    full = lambda shape: pl.BlockSpec(shape, lambda i: (0,) * len(shape))
    w_specs = [full((H, D, D)), full((H, 1, D)), full((H, 1, D)), full((H, 1, D)),
               full((D, D)), full((1, D)), full((1, D)), full((1, 1))]
    grid_spec = pl.GridSpec(
        grid=(NG // G,),
        in_specs=[pl.BlockSpec((3, 1, G * LP, 128), lambda i: (0, i, 0, 0)),
                  full((1, D)), full((1, D)), full((1, D)), full((1, 1))]
                 + w_specs + w_specs,
        out_specs=pl.BlockSpec((G, 1, D), lambda i: (i, 0, 0)),
    )
    pooled = pl.pallas_call(
        _main_kernel,
        grid_spec=grid_spec,
        out_shape=jax.ShapeDtypeStruct((NG, 1, D), jnp.float32),
    )(x, w_sim1.reshape(1, D), w_sim2.reshape(1, D),
      params["W_ws"].reshape(1, D), params["b_ws"].reshape(1, 1),
      *_mg_args(params["mg1"]), *_mg_args(params["mg2"]))

    return pl.pallas_call(
        _pearson_kernel,
        out_shape=jax.ShapeDtypeStruct((NG // 2,), jnp.float32),
    )(pooled)


# speedup vs baseline: 1.0847x; 1.0701x over previous
import functools

import jax
import jax.numpy as jnp
from jax import lax
from jax.experimental import pallas as pl
from jax.experimental.pallas import tpu as pltpu
from jax.experimental.pallas import tpu_sc as plsc

L = 50
LP = 56
D = 300
DP = 384
H = 4
G = 16
NG = 512

_NC = 2
_NS = 16
_NW = _NC * _NS
_NTOK = NG * LP
_BPW = _NTOK // _NW
_CH = 224
_NCHUNK = _BPW // _CH


@functools.lru_cache(maxsize=None)
def _make_gather_sc():
    @functools.partial(
        pl.kernel,
        mesh=plsc.VectorSubcoreMesh(core_axis_name="c", subcore_axis_name="s"),
        out_type=jax.ShapeDtypeStruct((_NTOK, DP), jnp.float32),
        scratch_types=[
            pltpu.VMEM((_CH,), jnp.int32),
            pltpu.VMEM((_CH, DP), jnp.float32),
            pltpu.SemaphoreType.DMA,
        ],
    )
    def _gather_sc(idx_hbm, table_hbm, out_hbm, idx_v, rows_v, sem):
        wid = lax.axis_index("s") * _NC + lax.axis_index("c")
        base = wid * _BPW
        for c in range(_NCHUNK):
            off = base + c * _CH
            pltpu.sync_copy(idx_hbm.at[pl.ds(off, _CH)], idx_v)
            pltpu.async_copy(table_hbm.at[idx_v], rows_v, sem).wait()
            pltpu.sync_copy(rows_v, out_hbm.at[pl.ds(off, _CH)])

    return _gather_sc


def _dot(a, b, dims):
    return jax.lax.dot_general(a, b, dimension_numbers=(dims, ((), ())),
                               preferred_element_type=jnp.float32)


def _round(x2d, eye, rowmask, wsim_gat, wsim_sage,
           Wg, al, ar, bg, Ws, bs, whw, bhw):
    zs = [_dot(x2d, Wg[h], ((1,), (0,))) for h in range(H)]
    e1 = jnp.maximum(x2d * wsim_gat, 0.0)
    e2 = jnp.maximum(x2d * wsim_sage, 0.0)
    ones_col = jnp.ones((LP, 1), jnp.float32)

    outs = []
    for g in range(G):
        s = slice(g * LP, (g + 1) * LP)
        xg = x2d[s]
        e1g = e1[s]
        mgat = (_dot(e1g, e1g, ((1,), (1,))) >= 0.1) | eye
        e2g = e2[s]
        msage = (_dot(e2g, e2g, ((1,), (1,))) >= 0.1) | eye

        chans = []
        for h in range(H):
            z = zs[h][s]
            el = _dot(z, al[h], ((1,), (1,)))
            er = _dot(ar[h], z, ((1,), (1,)))
            e = el + er
            e = jnp.where(e >= 0, e, 0.2 * e)
            e = jnp.where(mgat, e, -1e9)
            emax = jnp.max(e, axis=0, keepdims=True)
            ex = jnp.exp(e - emax)
            alpha = ex / jnp.sum(ex, axis=0, keepdims=True)
            att = _dot(alpha, z, ((0,), (0,)))
            chans.append(att + bg[h])

        m = msage.astype(jnp.float32)
        deg = _dot(m, ones_col, ((0,), (0,)))
        agg = _dot(m, xg, ((0,), (0,)))
        hs = (agg + xg) / (deg + 1.0)
        chans.append(_dot(hs, Ws, ((1,), (0,))) + bs)

        out = xg
        for ch in chans:
            gate = jax.nn.sigmoid(jnp.sum(ch * whw, axis=1, keepdims=True) + bhw)
            out = out + ch * gate
        outs.append(out)

    return jnp.concatenate(outs, axis=0) * rowmask


def _main_kernel(x_ref, ws1_ref, ws2_ref, wws_ref, bws_ref,
                 Wg1, al1, ar1, bg1, Ws1, bs1, whw1, bhw1,
                 Wg2, al2, ar2, bg2, Ws2, bs2, whw2, bhw2,
                 out_ref):
    x2d = x_ref[...].reshape(G * LP, DP)[:, 0:D]
    rowmask = (lax.broadcasted_iota(jnp.int32, (G * LP, 1), 0) % LP
               < L).astype(jnp.float32)
    x2d = x2d * rowmask
    iu = lax.broadcasted_iota(jnp.int32, (LP, LP), 0)
    iv = lax.broadcasted_iota(jnp.int32, (LP, LP), 1)
    eye = iu == iv
    ws1 = ws1_ref[...]
    ws2 = ws2_ref[...]
    x2d = _round(x2d, eye, rowmask, ws2, ws1, Wg1[...], al1[...], ar1[...],
                 bg1[...], Ws1[...], bs1[...], whw1[...], bhw1[...])
    x2d = _round(x2d, eye, rowmask, ws2, ws1, Wg2[...], al2[...], ar2[...],
                 bg2[...], Ws2[...], bs2[...], whw2[...], bhw2[...])
    gate = jax.nn.sigmoid(jnp.sum(x2d * wws_ref[...], axis=1, keepdims=True)
                          + bws_ref[...])
    xw = x2d * gate
    for g in range(G):
        out_ref[g] = jnp.sum(xw[g * LP:(g + 1) * LP], axis=0, keepdims=True)


def _pearson_kernel(p_ref, out_ref):
    p = p_ref[...].reshape(NG, D)
    g1 = p[0:NG // 2, :]
    g2 = p[NG // 2:NG, :]
    g1 = g1 - jnp.mean(g1, axis=1, keepdims=True)
    g2 = g2 - jnp.mean(g2, axis=1, keepdims=True)
    num = jnp.sum(g1 * g2, axis=1)
    den = jnp.sqrt(jnp.sum(g1 * g1, axis=1)) * jnp.sqrt(jnp.sum(g2 * g2, axis=1))
    out_ref[...] = num / den * 5.0


def _mg_args(p):
    Wg = p["W_gat"].reshape(D, H, D).transpose(1, 0, 2)
    al = p["attn_l"][:, None, :]
    ar = p["attn_r"][:, None, :]
    bg = p["b_gat"][:, None, :]
    whw = p["W_hw"].reshape(1, D)
    bhw = p["b_hw"].reshape(1, 1)
    return [Wg, al, ar, bg, p["W_sage"], p["b_sage"].reshape(1, D), whw, bhw]


def kernel(sentence_1, sentence_2, emb_table, w_sim1, w_sim2, params):
    idx2d = jnp.concatenate([sentence_1.T, sentence_2.T], axis=0)
    idx = jnp.pad(idx2d.astype(jnp.int32), ((0, 0), (0, LP - L))).reshape(-1)
    table_p = jnp.pad(emb_table, ((0, 0), (0, DP - D)))
    x = _make_gather_sc()(idx, table_p).reshape(NG // G, G * LP, DP)

    full = lambda shape: pl.BlockSpec(shape, lambda i: (0,) * len(shape))
    w_specs = [full((H, D, D)), full((H, 1, D)), full((H, 1, D)), full((H, 1, D)),
               full((D, D)), full((1, D)), full((1, D)), full((1, 1))]
    grid_spec = pl.GridSpec(
        grid=(NG // G,),
        in_specs=[pl.BlockSpec((1, G * LP, DP), lambda i: (i, 0, 0)),
                  full((1, D)), full((1, D)), full((1, D)), full((1, 1))]
                 + w_specs + w_specs,
        out_specs=pl.BlockSpec((G, 1, D), lambda i: (i, 0, 0)),
    )
    pooled = pl.pallas_call(
        _main_kernel,
        grid_spec=grid_spec,
        out_shape=jax.ShapeDtypeStruct((NG, 1, D), jnp.float32),
    )(x, w_sim1.reshape(1, D), w_sim2.reshape(1, D),
      params["W_ws"].reshape(1, D), params["b_ws"].reshape(1, 1),
      *_mg_args(params["mg1"]), *_mg_args(params["mg2"]))

    return pl.pallas_call(
        _pearson_kernel,
        out_shape=jax.ShapeDtypeStruct((NG // 2,), jnp.float32),
    )(pooled)
